# baseline (speedup 1.0000x reference)
import functools
import math

import jax
import jax.numpy as jnp
from jax.experimental import pallas as pl
from jax.experimental.pallas import tpu as pltpu

DIM = 1024
HEADS = 16
BLOCK = 128
DH = DIM // HEADS
SEQ = 2048
NMODS = 3

TB = 256
FTB = 512


def _block_attn_kernel(x_ref, wq_ref, wk_ref, wv_ref, wo_ref,
                       bq_ref, bk_ref, bv_ref, bo_ref, o_ref):
    f32 = jnp.float32
    bf16 = jnp.bfloat16
    x = x_ref[0]
    q = jnp.dot(x, wq_ref[0], preferred_element_type=f32) + bq_ref[0]
    k = jnp.dot(x, wk_ref[0], preferred_element_type=f32) + bk_ref[0]
    v = jnp.dot(x, wv_ref[0], preferred_element_type=f32) + bv_ref[0]
    qb = (q * (1.0 / math.sqrt(DH))).astype(bf16)
    kb = k.astype(bf16)
    vb = v.astype(bf16)
    nsb = TB // BLOCK
    row_blocks = []
    for s in range(nsb):
        qs = qb[s * BLOCK:(s + 1) * BLOCK]
        ks = kb[s * BLOCK:(s + 1) * BLOCK]
        vs = vb[s * BLOCK:(s + 1) * BLOCK]
        heads = []
        for h in range(HEADS):
            qh = qs[:, h * DH:(h + 1) * DH]
            kh = ks[:, h * DH:(h + 1) * DH]
            vh = vs[:, h * DH:(h + 1) * DH]
            sc = jax.lax.dot_general(
                qh, kh, (((1,), (1,)), ((), ())),
                preferred_element_type=f32)
            m = jnp.max(sc, axis=-1, keepdims=True)
            e = jnp.exp(sc - m)
            p = e / jnp.sum(e, axis=-1, keepdims=True)
            heads.append(jnp.dot(p.astype(bf16), vh,
                                 preferred_element_type=f32))
        row_blocks.append(jnp.concatenate(heads, axis=-1))
    att = jnp.concatenate(row_blocks, axis=0)
    o = jnp.dot(att.astype(bf16), wo_ref[0],
                preferred_element_type=f32) + bo_ref[0]
    o_ref[0] = o.astype(o_ref.dtype)


def _block_attn(x, wq, wk, wv, wo, bq, bk, bv, bo):
    m, s, _ = x.shape
    ntb = s // TB
    wspec = pl.BlockSpec((1, DIM, DIM), lambda i, j: (i, 0, 0))
    bspec = pl.BlockSpec((1, 1, DIM), lambda i, j: (i, 0, 0))
    return pl.pallas_call(
        _block_attn_kernel,
        grid=(m, ntb),
        in_specs=[
            pl.BlockSpec((1, TB, DIM), lambda i, j: (i, j, 0)),
            wspec, wspec, wspec, wspec,
            bspec, bspec, bspec, bspec,
        ],
        out_specs=pl.BlockSpec((1, TB, DIM), lambda i, j: (i, j, 0)),
        out_shape=jax.ShapeDtypeStruct((m, s, DIM), jnp.bfloat16),
    )(x, wq, wk, wv, wo, bq, bk, bv, bo)


def _fusion_kernel(a_ref, c_ref, w_ref, b_ref, o_ref):
    f32 = jnp.float32
    acc = jnp.dot(a_ref[0], w_ref[0], preferred_element_type=f32)
    for i in range(1, NMODS):
        acc += jnp.dot(a_ref[i], w_ref[i], preferred_element_type=f32)
    for i in range(NMODS):
        acc += jnp.dot(c_ref[i], w_ref[NMODS + i], preferred_element_type=f32)
    o_ref[...] = acc + b_ref[...]


def _fusion(a, c, wf, bf):
    nt = SEQ // FTB
    return pl.pallas_call(
        _fusion_kernel,
        grid=(nt,),
        in_specs=[
            pl.BlockSpec((NMODS, FTB, DIM), lambda i: (0, i, 0)),
            pl.BlockSpec((NMODS, FTB, DIM), lambda i: (0, i, 0)),
            pl.BlockSpec((2 * NMODS, DIM, DIM), lambda i: (0, 0, 0)),
            pl.BlockSpec((1, DIM), lambda i: (0, 0)),
        ],
        out_specs=pl.BlockSpec((FTB, DIM), lambda i: (i, 0)),
        out_shape=jax.ShapeDtypeStruct((SEQ, DIM), jnp.float32),
    )(a, c, wf, bf)


def _stack_params(params, names, dtype):
    return [jnp.stack([params[m + "_attn"][n] for m in ("text", "visual", "audio")]).astype(dtype)
            for n in names]


def kernel(text, visual, audio, params):
    bf16 = jnp.bfloat16
    x = jnp.stack([text[0], visual[0], audio[0]]).astype(bf16)
    wq, wk, wv, wo = _stack_params(params, ("Wq", "Wk", "Wv", "Wo"), bf16)
    bq, bk, bv, bo = [b.reshape(NMODS, 1, DIM)
                      for b in _stack_params(params, ("bq", "bk", "bv", "bo"),
                                             jnp.float32)]
    attended = _block_attn(x, wq, wk, wv, wo, bq, bk, bv, bo)

    cp = params["cross_attn"]
    cw = [cp[n].astype(bf16)[None] for n in ("Wq", "Wk", "Wv", "Wo")]
    cb = [cp[n].astype(jnp.float32).reshape(1, 1, DIM)
          for n in ("bq", "bk", "bv", "bo")]
    cross = _block_attn(attended.reshape(1, NMODS * SEQ, DIM), *cw, *cb)
    cross = cross.reshape(NMODS, SEQ, DIM)

    fw = params["fusion_weights"].astype(jnp.float32)
    scales = jnp.concatenate([fw, fw]).reshape(2 * NMODS, 1, 1)
    wf = (params["fusion_W"].reshape(2 * NMODS, DIM, DIM) * scales).astype(bf16)
    bfus = params["fusion_b"].astype(jnp.float32).reshape(1, DIM)
    out = _fusion(attended, cross, wf, bfus)
    return out.reshape(1, SEQ, DIM)


# speedup vs baseline: 2.1975x; 2.1975x over previous
import functools
import math

import jax
import jax.numpy as jnp
from jax.experimental import pallas as pl
from jax.experimental.pallas import tpu as pltpu

DIM = 1024
HEADS = 16
BLOCK = 128
DH = DIM // HEADS
SEQ = 2048
NMODS = 3

TB = 256
FTB = 512


def _block_attn_kernel(x_ref, wq_ref, wk_ref, wv_ref, wo_ref,
                       bq_ref, bk_ref, bv_ref, bo_ref, o_ref):
    f32 = jnp.float32
    bf16 = jnp.bfloat16
    x = x_ref[0]
    q = jnp.dot(x, wq_ref[0], preferred_element_type=f32) + bq_ref[0]
    k = jnp.dot(x, wk_ref[0], preferred_element_type=f32) + bk_ref[0]
    v = jnp.dot(x, wv_ref[0], preferred_element_type=f32) + bv_ref[0]
    qb = (q * (1.0 / math.sqrt(DH))).astype(bf16)
    kb = k.astype(bf16)
    vb = v.astype(bf16)
    nsb = TB // BLOCK
    scores = []
    for s in range(nsb):
        qs = qb[s * BLOCK:(s + 1) * BLOCK]
        ks = kb[s * BLOCK:(s + 1) * BLOCK]
        for h in range(HEADS):
            qh = qs[:, h * DH:(h + 1) * DH]
            kh = ks[:, h * DH:(h + 1) * DH]
            scores.append(jax.lax.dot_general(
                qh, kh, (((1,), (1,)), ((), ())),
                preferred_element_type=f32))
    sc = jnp.concatenate(scores, axis=0)
    m = jnp.max(sc, axis=-1, keepdims=True)
    e = jnp.exp(sc - m)
    p = e / jnp.sum(e, axis=-1, keepdims=True)
    pb = p.astype(bf16)
    row_blocks = []
    for s in range(nsb):
        vs = vb[s * BLOCK:(s + 1) * BLOCK]
        heads = []
        for h in range(HEADS):
            ph = pb[(s * HEADS + h) * BLOCK:(s * HEADS + h + 1) * BLOCK]
            vh = vs[:, h * DH:(h + 1) * DH]
            heads.append(jnp.dot(ph, vh, preferred_element_type=f32))
        row_blocks.append(jnp.concatenate(heads, axis=-1))
    att = jnp.concatenate(row_blocks, axis=0)
    o = jnp.dot(att.astype(bf16), wo_ref[0],
                preferred_element_type=f32) + bo_ref[0]
    o_ref[0] = o.astype(o_ref.dtype)


def _block_attn(x, wq, wk, wv, wo, bq, bk, bv, bo):
    m, s, _ = x.shape
    ntb = s // TB
    wspec = pl.BlockSpec((1, DIM, DIM), lambda i, j: (i, 0, 0))
    bspec = pl.BlockSpec((1, 1, DIM), lambda i, j: (i, 0, 0))
    return pl.pallas_call(
        _block_attn_kernel,
        grid=(m, ntb),
        in_specs=[
            pl.BlockSpec((1, TB, DIM), lambda i, j: (i, j, 0)),
            wspec, wspec, wspec, wspec,
            bspec, bspec, bspec, bspec,
        ],
        out_specs=pl.BlockSpec((1, TB, DIM), lambda i, j: (i, j, 0)),
        out_shape=jax.ShapeDtypeStruct((m, s, DIM), jnp.bfloat16),
    )(x, wq, wk, wv, wo, bq, bk, bv, bo)


def _fusion_kernel(a_ref, c_ref, w_ref, b_ref, o_ref):
    f32 = jnp.float32
    acc = jnp.dot(a_ref[0], w_ref[0], preferred_element_type=f32)
    for i in range(1, NMODS):
        acc += jnp.dot(a_ref[i], w_ref[i], preferred_element_type=f32)
    for i in range(NMODS):
        acc += jnp.dot(c_ref[i], w_ref[NMODS + i], preferred_element_type=f32)
    o_ref[...] = acc + b_ref[...]


def _fusion(a, c, wf, bf):
    nt = SEQ // FTB
    return pl.pallas_call(
        _fusion_kernel,
        grid=(nt,),
        in_specs=[
            pl.BlockSpec((NMODS, FTB, DIM), lambda i: (0, i, 0)),
            pl.BlockSpec((NMODS, FTB, DIM), lambda i: (0, i, 0)),
            pl.BlockSpec((2 * NMODS, DIM, DIM), lambda i: (0, 0, 0)),
            pl.BlockSpec((1, DIM), lambda i: (0, 0)),
        ],
        out_specs=pl.BlockSpec((FTB, DIM), lambda i: (i, 0)),
        out_shape=jax.ShapeDtypeStruct((SEQ, DIM), jnp.float32),
    )(a, c, wf, bf)


def _stack_params(params, names, dtype):
    return [jnp.stack([params[m + "_attn"][n] for m in ("text", "visual", "audio")]).astype(dtype)
            for n in names]


def kernel(text, visual, audio, params):
    bf16 = jnp.bfloat16
    x = jnp.stack([text[0], visual[0], audio[0]]).astype(bf16)
    wq, wk, wv, wo = _stack_params(params, ("Wq", "Wk", "Wv", "Wo"), bf16)
    bq, bk, bv, bo = [b.reshape(NMODS, 1, DIM)
                      for b in _stack_params(params, ("bq", "bk", "bv", "bo"),
                                             jnp.float32)]
    attended = _block_attn(x, wq, wk, wv, wo, bq, bk, bv, bo)

    cp = params["cross_attn"]
    cw = [cp[n].astype(bf16)[None] for n in ("Wq", "Wk", "Wv", "Wo")]
    cb = [cp[n].astype(jnp.float32).reshape(1, 1, DIM)
          for n in ("bq", "bk", "bv", "bo")]
    cross = _block_attn(attended.reshape(1, NMODS * SEQ, DIM), *cw, *cb)
    cross = cross.reshape(NMODS, SEQ, DIM)

    fw = params["fusion_weights"].astype(jnp.float32)
    scales = jnp.concatenate([fw, fw]).reshape(2 * NMODS, 1, 1)
    wf = (params["fusion_W"].reshape(2 * NMODS, DIM, DIM) * scales).astype(bf16)
    bfus = params["fusion_b"].astype(jnp.float32).reshape(1, DIM)
    out = _fusion(attended, cross, wf, bfus)
    return out.reshape(1, SEQ, DIM)
